# baseline (speedup 1.0000x reference)
import functools

import jax
import jax.numpy as jnp
from jax import lax
from jax.experimental import pallas as pl
from jax.experimental.pallas import tpu as pltpu
from jax.experimental.pallas import tpu_sc as plsc

EMB = 100000
DIM = 128
D2 = 2 * DIM
BATCH = 16384

NC = 2
NS = 16
NW = NC * NS
BPW = BATCH // NW
CHUNK = 128


def _sc_gather(pos_u, pos_w, W_word, W_ctx):
    mesh = plsc.VectorSubcoreMesh(
        core_axis_name="c", subcore_axis_name="s", num_cores=NC, num_subcores=NS
    )

    @functools.partial(
        pl.kernel,
        mesh=mesh,
        out_type=(
            jax.ShapeDtypeStruct((BATCH, D2), jnp.float32),
            jax.ShapeDtypeStruct((BATCH, D2), jnp.float32),
        ),
        scratch_types=[
            pltpu.VMEM((CHUNK,), jnp.int32),
            pltpu.VMEM((CHUNK,), jnp.int32),
            pltpu.VMEM((CHUNK, D2), jnp.float32),
            pltpu.VMEM((CHUNK, D2), jnp.float32),
            pltpu.SemaphoreType.DMA,
            pltpu.SemaphoreType.DMA,
        ],
    )
    def k(pu_hbm, pw_hbm, wu_hbm, wc_hbm, gu_hbm, gw_hbm,
          idxu_v, idxw_v, rowsu_v, rowsw_v, semu, semw):
        wid = lax.axis_index("s") * NC + lax.axis_index("c")
        base = wid * BPW

        def body(i, carry):
            off = base + i * CHUNK
            pltpu.sync_copy(pu_hbm.at[pl.ds(off, CHUNK)], idxu_v)
            pltpu.sync_copy(pw_hbm.at[pl.ds(off, CHUNK)], idxw_v)
            cu = pltpu.async_copy(wu_hbm.at[idxu_v], rowsu_v, semu)
            cw = pltpu.async_copy(wc_hbm.at[idxw_v], rowsw_v, semw)
            cu.wait()
            cw.wait()
            pltpu.sync_copy(rowsu_v, gu_hbm.at[pl.ds(off, CHUNK)])
            pltpu.sync_copy(rowsw_v, gw_hbm.at[pl.ds(off, CHUNK)])
            return carry

        lax.fori_loop(0, BPW // CHUNK, body, 0)

    return k(pos_u, pos_w, W_word, W_ctx)


def _tc_body(gu_ref, gw_ref, tv_ref, pv_ref, iv_ref):
    u = gu_ref[...]
    w = gw_ref[...]
    zu = u[:, :DIM]
    su = jax.nn.softplus(u[:, DIM:])
    zw = w[:, :DIM]
    sw = jax.nn.softplus(w[:, DIM:])
    tv_ref[...] = jnp.sum(jnp.log(jax.nn.softplus(su) + 1e-23), axis=-1, keepdims=True)
    pv_ref[...] = jnp.sum(jnp.log(jax.nn.softplus(sw) + 1e-23), axis=-1, keepdims=True)
    ti = jnp.minimum(zu + su, zw + sw) - jnp.maximum(zu, zw)
    iv_ref[...] = jnp.sum(jnp.log(jax.nn.softplus(ti) + 1e-23), axis=-1, keepdims=True)


_TC_ROWS = 256


def _tc_compute(gu, gw):
    grid = (BATCH // _TC_ROWS,)
    outs = pl.pallas_call(
        _tc_body,
        grid=grid,
        in_specs=[pl.BlockSpec((_TC_ROWS, D2), lambda i: (i, 0))] * 2,
        out_specs=[pl.BlockSpec((_TC_ROWS, 1), lambda i: (i, 0))] * 3,
        out_shape=[jax.ShapeDtypeStruct((BATCH, 1), jnp.float32)] * 3,
    )(gu, gw)
    return tuple(o.reshape(BATCH) for o in outs)


def kernel(pos_u, pos_w, W_word, W_ctx):
    gu, gw = _sc_gather(pos_u, pos_w, W_word, W_ctx)
    return _tc_compute(gu, gw)
